# baseline (speedup 1.0000x reference)
import functools

import jax
import jax.numpy as jnp
from jax import lax
from jax.experimental import pallas as pl
from jax.experimental.pallas import tpu as pltpu
from jax.experimental.pallas import tpu_sc as plsc

_NUM_EMB = 90000
_EMB_DIM = 32
_HDIM = _EMB_DIM // 2
_BATCH = 16384

_info = plsc.get_sparse_core_info()
_NC = _info.num_cores
_NS = _info.num_subcores
_L = _info.num_lanes
_NW = _NC * _NS
_BPW = _BATCH // _NW
_ICHUNK = 128
_NICHUNK = _BPW // _ICHUNK
_NGRP = _ICHUNK // _L
_HBUF = _HDIM * _ICHUNK


def _stage_indices(uid_hbm, iid_hbm, uidx_v, iidx_v, base, sem_idx):
    idx_copies = []
    for j in range(_NICHUNK):
        idx_copies.append(pltpu.async_copy(
            uid_hbm.at[pl.ds(base + j * _ICHUNK, _ICHUNK)],
            uidx_v.at[j], sem_idx))
        idx_copies.append(pltpu.async_copy(
            iid_hbm.at[pl.ds(base + j * _ICHUNK, _ICHUNK)],
            iidx_v.at[j], sem_idx))
    for c in idx_copies:
        c.wait()


def _fire(j, ut_hbm, it_hbm, uidx_v, iidx_v, ubuf_v, ibuf_v, sem_u, sem_i):
    buf = j % 2

    def fire_dim(d, carry):
        src_u = ut_hbm.at[pl.ds(d * _NUM_EMB, _NUM_EMB)]
        src_i = it_hbm.at[pl.ds(d * _NUM_EMB, _NUM_EMB)]
        off = buf * _HBUF + d * _ICHUNK
        pltpu.async_copy(src_u.at[uidx_v.at[j]],
                         ubuf_v.at[pl.ds(off, _ICHUNK)], sem_u)
        pltpu.async_copy(src_i.at[iidx_v.at[j]],
                         ibuf_v.at[pl.ds(off, _ICHUNK)], sem_i)
        return carry

    lax.fori_loop(0, _HDIM, fire_dim, 0)


def _drain(j, ut_hbm, it_hbm, ubuf_v, ibuf_v, sem_u, sem_i):
    buf = j % 2
    pltpu.make_async_copy(ut_hbm.at[pl.ds(0, _HBUF)],
                          ubuf_v.at[pl.ds(buf * _HBUF, _HBUF)], sem_u).wait()
    pltpu.make_async_copy(it_hbm.at[pl.ds(0, _HBUF)],
                          ibuf_v.at[pl.ds(buf * _HBUF, _HBUF)], sem_i).wait()


def _accumulate(j, ubuf_v, ibuf_v, g):
    buf = j % 2
    acc_d = jnp.zeros((_L,), jnp.float32)
    acc_u = jnp.zeros((_L,), jnp.float32)
    acc_i = jnp.zeros((_L,), jnp.float32)
    for d in range(_HDIM):
        s = pl.ds(buf * _HBUF + d * _ICHUNK + g * _L, _L)
        uc = ubuf_v[s]
        ic = ibuf_v[s]
        acc_d = acc_d + uc * ic
        acc_u = acc_u + uc * uc
        acc_i = acc_i + ic * ic
    return acc_d, acc_u, acc_i


_SCRATCH = [
    pltpu.VMEM((_NICHUNK, _ICHUNK), jnp.int32),
    pltpu.VMEM((_NICHUNK, _ICHUNK), jnp.int32),
    pltpu.VMEM((2 * _HBUF,), jnp.float32),
    pltpu.VMEM((2 * _HBUF,), jnp.float32),
    pltpu.VMEM((_BPW,), jnp.float32),
    pltpu.VMEM((_BPW,), jnp.float32),
    pltpu.VMEM((_BPW,), jnp.float32),
    pltpu.SemaphoreType.DMA,
    pltpu.SemaphoreType.DMA,
    pltpu.SemaphoreType.DMA,
]

_PARAMS = pltpu.CompilerParams(needs_layout_passes=False)
_MESH = plsc.VectorSubcoreMesh(core_axis_name="c", subcore_axis_name="s")


@functools.partial(
    pl.kernel, mesh=_MESH,
    out_type=(jax.ShapeDtypeStruct((_BATCH,), jnp.float32),
              jax.ShapeDtypeStruct((_BATCH,), jnp.float32),
              jax.ShapeDtypeStruct((_BATCH,), jnp.float32)),
    scratch_types=_SCRATCH,
    compiler_params=_PARAMS,
)
def _sc_partial(uid_hbm, iid_hbm, ut_hbm, it_hbm,
                pd_hbm, pu_hbm, pi_hbm,
                uidx_v, iidx_v, ubuf_v, ibuf_v, pd_v, pu_v, pi_v,
                sem_idx, sem_u, sem_i):
    wid = lax.axis_index("s") * _NC + lax.axis_index("c")
    base = wid * _BPW
    _stage_indices(uid_hbm, iid_hbm, uidx_v, iidx_v, base, sem_idx)

    _fire(0, ut_hbm, it_hbm, uidx_v, iidx_v, ubuf_v, ibuf_v, sem_u, sem_i)
    for j in range(_NICHUNK):
        if j + 1 < _NICHUNK:
            _fire(j + 1, ut_hbm, it_hbm, uidx_v, iidx_v, ubuf_v, ibuf_v,
                  sem_u, sem_i)
        _drain(j, ut_hbm, it_hbm, ubuf_v, ibuf_v, sem_u, sem_i)

        def group_body(g, carry):
            acc_d, acc_u, acc_i = _accumulate(j, ubuf_v, ibuf_v, g)
            s = pl.ds(j * _ICHUNK + g * _L, _L)
            pd_v[s] = acc_d
            pu_v[s] = acc_u
            pi_v[s] = acc_i
            return carry

        lax.fori_loop(0, _NGRP, group_body, 0)

    pltpu.sync_copy(pd_v, pd_hbm.at[pl.ds(base, _BPW)])
    pltpu.sync_copy(pu_v, pu_hbm.at[pl.ds(base, _BPW)])
    pltpu.sync_copy(pi_v, pi_hbm.at[pl.ds(base, _BPW)])


@functools.partial(
    pl.kernel, mesh=_MESH,
    out_type=jax.ShapeDtypeStruct((_BATCH,), jnp.float32),
    scratch_types=_SCRATCH + [pltpu.VMEM((_BPW,), jnp.float32)] * 2,
    compiler_params=_PARAMS,
)
def _sc_final(uid_hbm, iid_hbm, ut_hbm, it_hbm, pd_hbm, pu_hbm, pi_hbm,
              out_hbm,
              uidx_v, iidx_v, ubuf_v, ibuf_v, out_v, pu_v, pi_v,
              sem_idx, sem_u, sem_i, pd_v, spare_v):
    wid = lax.axis_index("s") * _NC + lax.axis_index("c")
    base = wid * _BPW
    _stage_indices(uid_hbm, iid_hbm, uidx_v, iidx_v, base, sem_idx)

    p_copies = [
        pltpu.async_copy(pd_hbm.at[pl.ds(base, _BPW)], pd_v, sem_idx),
        pltpu.async_copy(pu_hbm.at[pl.ds(base, _BPW)], pu_v, sem_idx),
        pltpu.async_copy(pi_hbm.at[pl.ds(base, _BPW)], pi_v, sem_idx),
    ]

    _fire(0, ut_hbm, it_hbm, uidx_v, iidx_v, ubuf_v, ibuf_v, sem_u, sem_i)
    for c in p_copies:
        c.wait()
    for j in range(_NICHUNK):
        if j + 1 < _NICHUNK:
            _fire(j + 1, ut_hbm, it_hbm, uidx_v, iidx_v, ubuf_v, ibuf_v,
                  sem_u, sem_i)
        _drain(j, ut_hbm, it_hbm, ubuf_v, ibuf_v, sem_u, sem_i)

        def group_body(g, carry):
            acc_d, acc_u, acc_i = _accumulate(j, ubuf_v, ibuf_v, g)
            s = pl.ds(j * _ICHUNK + g * _L, _L)
            acc_d = acc_d + pd_v[s]
            acc_u = acc_u + pu_v[s]
            acc_i = acc_i + pi_v[s]
            p = jnp.maximum(acc_u, 1e-16) * jnp.maximum(acc_i, 1e-16)
            seed = jnp.full((_L,), 0x5F3759DF, jnp.int32) - \
                lax.shift_right_logical(plsc.bitcast(p, jnp.int32), 1)
            y = plsc.bitcast(seed, jnp.float32)
            for _ in range(3):
                y = y * (1.5 - 0.5 * p * y * y)
            out_v[s] = (6.0 * acc_d) * y
            return carry

        lax.fori_loop(0, _NGRP, group_body, 0)

    pltpu.sync_copy(out_v, out_hbm.at[pl.ds(base, _BPW)])


def kernel(user_id, item_id, user_table, item_table):
    uid = user_id.astype(jnp.int32)
    iid = item_id.astype(jnp.int32)
    u0 = user_table[:, :_HDIM].T.reshape(_HDIM * _NUM_EMB)
    i0 = item_table[:, :_HDIM].T.reshape(_HDIM * _NUM_EMB)
    u1 = user_table[:, _HDIM:].T.reshape(_HDIM * _NUM_EMB)
    i1 = item_table[:, _HDIM:].T.reshape(_HDIM * _NUM_EMB)
    pd, pu, pi = _sc_partial(uid, iid, u0, i0)
    return _sc_final(uid, iid, u1, i1, pd, pu, pi)


# speedup vs baseline: 1.2143x; 1.2143x over previous
import functools

import jax
import jax.numpy as jnp
from jax import lax
from jax.experimental import pallas as pl
from jax.experimental.pallas import tpu as pltpu
from jax.experimental.pallas import tpu_sc as plsc

_NUM_EMB = 90000
_EMB_DIM = 32
_BATCH = 16384
_FLAT = _NUM_EMB * _EMB_DIM

_info = plsc.get_sparse_core_info()
_NC = _info.num_cores
_NS = _info.num_subcores
_L = _info.num_lanes
_NW = _NC * _NS
_BPW = _BATCH // _NW
_ICHUNK = 128
_NICHUNK = _BPW // _ICHUNK
_WBUF = _EMB_DIM * _BPW

_MESH = plsc.VectorSubcoreMesh(core_axis_name="c", subcore_axis_name="s")
_PARAMS = pltpu.CompilerParams(needs_layout_passes=False)


def _stage_idx(idx_hbm, idx_v, base, sem):
    cs = [pltpu.async_copy(idx_hbm.at[pl.ds(base + j * _ICHUNK, _ICHUNK)],
                           idx_v.at[j], sem)
          for j in range(_NICHUNK)]
    for c in cs:
        c.wait()


def _gather_all(t_hbm, idx_v, buf_v, sem):
    def fire_dim(d, carry):
        src = t_hbm.at[pl.ds(d * _NUM_EMB, _NUM_EMB)]
        for j in range(_NICHUNK):
            off = d * _BPW + j * _ICHUNK
            pltpu.async_copy(src.at[idx_v.at[j]],
                             buf_v.at[pl.ds(off, _ICHUNK)], sem)
        return carry

    lax.fori_loop(0, _EMB_DIM, fire_dim, 0)
    pltpu.make_async_copy(t_hbm.at[pl.ds(0, _WBUF)], buf_v, sem).wait()


@functools.partial(
    pl.kernel, mesh=_MESH,
    out_type=(jax.ShapeDtypeStruct((_NW * _WBUF,), jnp.float32),
              jax.ShapeDtypeStruct((_BATCH,), jnp.float32)),
    scratch_types=[
        pltpu.VMEM((_NICHUNK, _ICHUNK), jnp.int32),
        pltpu.VMEM((_WBUF,), jnp.float32),
        pltpu.VMEM((_BPW,), jnp.float32),
        pltpu.SemaphoreType.DMA,
        pltpu.SemaphoreType.DMA,
    ],
    compiler_params=_PARAMS,
)
def _sc_user(uid_hbm, ut_hbm, stash_hbm, pu_hbm,
             uidx_v, ubuf_v, pu_v, sem_idx, sem_g):
    wid = lax.axis_index("s") * _NC + lax.axis_index("c")
    base = wid * _BPW
    _stage_idx(uid_hbm, uidx_v, base, sem_idx)
    _gather_all(ut_hbm, uidx_v, ubuf_v, sem_g)

    def group_body(g, carry):
        acc_u = jnp.zeros((_L,), jnp.float32)
        for d in range(_EMB_DIM):
            uc = ubuf_v[pl.ds(d * _BPW + g * _L, _L)]
            acc_u = acc_u + uc * uc
        pu_v[pl.ds(g * _L, _L)] = acc_u
        return carry

    lax.fori_loop(0, _BPW // _L, group_body, 0)

    pltpu.sync_copy(ubuf_v, stash_hbm.at[pl.ds(wid * _WBUF, _WBUF)])
    pltpu.sync_copy(pu_v, pu_hbm.at[pl.ds(base, _BPW)])


@functools.partial(
    pl.kernel, mesh=_MESH,
    out_type=jax.ShapeDtypeStruct((_BATCH,), jnp.float32),
    scratch_types=[
        pltpu.VMEM((_NICHUNK, _ICHUNK), jnp.int32),
        pltpu.VMEM((_WBUF,), jnp.float32),
        pltpu.VMEM((_WBUF,), jnp.float32),
        pltpu.VMEM((_BPW,), jnp.float32),
        pltpu.VMEM((_BPW,), jnp.float32),
        pltpu.SemaphoreType.DMA,
        pltpu.SemaphoreType.DMA,
    ],
    compiler_params=_PARAMS,
)
def _sc_item(iid_hbm, it_hbm, stash_hbm, pu_hbm, out_hbm,
             iidx_v, ibuf_v, ubuf_v, pu_v, out_v, sem_idx, sem_g):
    wid = lax.axis_index("s") * _NC + lax.axis_index("c")
    base = wid * _BPW
    _stage_idx(iid_hbm, iidx_v, base, sem_idx)
    stash_cp = pltpu.async_copy(stash_hbm.at[pl.ds(wid * _WBUF, _WBUF)],
                                ubuf_v, sem_idx)
    pu_cp = pltpu.async_copy(pu_hbm.at[pl.ds(base, _BPW)], pu_v, sem_idx)
    _gather_all(it_hbm, iidx_v, ibuf_v, sem_g)
    stash_cp.wait()
    pu_cp.wait()

    def group_body(g, carry):
        acc_d = jnp.zeros((_L,), jnp.float32)
        acc_i = jnp.zeros((_L,), jnp.float32)
        for d in range(_EMB_DIM):
            s = pl.ds(d * _BPW + g * _L, _L)
            ic = ibuf_v[s]
            uc = ubuf_v[s]
            acc_d = acc_d + uc * ic
            acc_i = acc_i + ic * ic
        acc_u = pu_v[pl.ds(g * _L, _L)]
        p = jnp.maximum(acc_u, 1e-16) * jnp.maximum(acc_i, 1e-16)
        seed = jnp.full((_L,), 0x5F3759DF, jnp.int32) - \
            lax.shift_right_logical(plsc.bitcast(p, jnp.int32), 1)
        y = plsc.bitcast(seed, jnp.float32)
        for _ in range(3):
            y = y * (1.5 - 0.5 * p * y * y)
        out_v[pl.ds(g * _L, _L)] = (6.0 * acc_d) * y
        return carry

    lax.fori_loop(0, _BPW // _L, group_body, 0)

    pltpu.sync_copy(out_v, out_hbm.at[pl.ds(base, _BPW)])


def kernel(user_id, item_id, user_table, item_table):
    uid = user_id.astype(jnp.int32)
    iid = item_id.astype(jnp.int32)
    uflat = user_table.T.reshape(_FLAT)
    iflat = item_table.T.reshape(_FLAT)
    stash, pu = _sc_user(uid, uflat)
    return _sc_item(iid, iflat, stash, pu)


# speedup vs baseline: 1.2190x; 1.0039x over previous
import functools

import jax
import jax.numpy as jnp
from jax import lax
from jax.experimental import pallas as pl
from jax.experimental.pallas import tpu as pltpu
from jax.experimental.pallas import tpu_sc as plsc

_NUM_EMB = 90000
_EMB_DIM = 32
_BATCH = 16384
_FLAT = _NUM_EMB * _EMB_DIM

_info = plsc.get_sparse_core_info()
_NC = _info.num_cores
_NS = _info.num_subcores
_L = _info.num_lanes
_NW = _NC * _NS
_BPW = _BATCH // _NW
_ICHUNK = 128
_NICHUNK = _BPW // _ICHUNK
_WBUF = _EMB_DIM * _BPW

_MESH = plsc.VectorSubcoreMesh(core_axis_name="c", subcore_axis_name="s")
_PARAMS = pltpu.CompilerParams(needs_layout_passes=False)


def _stage_idx(idx_hbm, idx_v, base, sem):
    cp = pltpu.async_copy(idx_hbm.at[pl.ds(base, _BPW)], idx_v, sem)
    cp.wait()


def _gather_all(t_hbm, idx_v, buf_v, sem):
    def fire_dim(d, carry):
        src = t_hbm.at[pl.ds(d * _NUM_EMB, _NUM_EMB)]
        pltpu.async_copy(src.at[idx_v],
                         buf_v.at[pl.ds(d * _BPW, _BPW)], sem)
        return carry

    lax.fori_loop(0, _EMB_DIM, fire_dim, 0)
    pltpu.make_async_copy(t_hbm.at[pl.ds(0, _WBUF)], buf_v, sem).wait()


@functools.partial(
    pl.kernel, mesh=_MESH,
    out_type=(jax.ShapeDtypeStruct((_NW * _WBUF,), jnp.float32),
              jax.ShapeDtypeStruct((_BATCH,), jnp.float32)),
    scratch_types=[
        pltpu.VMEM((_BPW,), jnp.int32),
        pltpu.VMEM((_WBUF,), jnp.float32),
        pltpu.VMEM((_BPW,), jnp.float32),
        pltpu.SemaphoreType.DMA,
        pltpu.SemaphoreType.DMA,
    ],
    compiler_params=_PARAMS,
)
def _sc_user(uid_hbm, ut_hbm, stash_hbm, pu_hbm,
             uidx_v, ubuf_v, pu_v, sem_idx, sem_g):
    wid = lax.axis_index("s") * _NC + lax.axis_index("c")
    base = wid * _BPW
    _stage_idx(uid_hbm, uidx_v, base, sem_idx)
    _gather_all(ut_hbm, uidx_v, ubuf_v, sem_g)

    def group_body(g, carry):
        acc_u = jnp.zeros((_L,), jnp.float32)
        for d in range(_EMB_DIM):
            uc = ubuf_v[pl.ds(d * _BPW + g * _L, _L)]
            acc_u = acc_u + uc * uc
        pu_v[pl.ds(g * _L, _L)] = acc_u
        return carry

    lax.fori_loop(0, _BPW // _L, group_body, 0)

    pltpu.sync_copy(ubuf_v, stash_hbm.at[pl.ds(wid * _WBUF, _WBUF)])
    pltpu.sync_copy(pu_v, pu_hbm.at[pl.ds(base, _BPW)])


@functools.partial(
    pl.kernel, mesh=_MESH,
    out_type=jax.ShapeDtypeStruct((_BATCH,), jnp.float32),
    scratch_types=[
        pltpu.VMEM((_BPW,), jnp.int32),
        pltpu.VMEM((_WBUF,), jnp.float32),
        pltpu.VMEM((_WBUF,), jnp.float32),
        pltpu.VMEM((_BPW,), jnp.float32),
        pltpu.VMEM((_BPW,), jnp.float32),
        pltpu.SemaphoreType.DMA,
        pltpu.SemaphoreType.DMA,
    ],
    compiler_params=_PARAMS,
)
def _sc_item(iid_hbm, it_hbm, stash_hbm, pu_hbm, out_hbm,
             iidx_v, ibuf_v, ubuf_v, pu_v, out_v, sem_idx, sem_g):
    wid = lax.axis_index("s") * _NC + lax.axis_index("c")
    base = wid * _BPW
    _stage_idx(iid_hbm, iidx_v, base, sem_idx)
    stash_cp = pltpu.async_copy(stash_hbm.at[pl.ds(wid * _WBUF, _WBUF)],
                                ubuf_v, sem_idx)
    pu_cp = pltpu.async_copy(pu_hbm.at[pl.ds(base, _BPW)], pu_v, sem_idx)
    _gather_all(it_hbm, iidx_v, ibuf_v, sem_g)
    stash_cp.wait()
    pu_cp.wait()

    def group_body(g, carry):
        acc_d = jnp.zeros((_L,), jnp.float32)
        acc_i = jnp.zeros((_L,), jnp.float32)
        for d in range(_EMB_DIM):
            s = pl.ds(d * _BPW + g * _L, _L)
            ic = ibuf_v[s]
            uc = ubuf_v[s]
            acc_d = acc_d + uc * ic
            acc_i = acc_i + ic * ic
        acc_u = pu_v[pl.ds(g * _L, _L)]
        p = jnp.maximum(acc_u, 1e-16) * jnp.maximum(acc_i, 1e-16)
        seed = jnp.full((_L,), 0x5F3759DF, jnp.int32) - \
            lax.shift_right_logical(plsc.bitcast(p, jnp.int32), 1)
        y = plsc.bitcast(seed, jnp.float32)
        for _ in range(3):
            y = y * (1.5 - 0.5 * p * y * y)
        out_v[pl.ds(g * _L, _L)] = (6.0 * acc_d) * y
        return carry

    lax.fori_loop(0, _BPW // _L, group_body, 0)

    pltpu.sync_copy(out_v, out_hbm.at[pl.ds(base, _BPW)])


def kernel(user_id, item_id, user_table, item_table):
    uid = user_id.astype(jnp.int32)
    iid = item_id.astype(jnp.int32)
    uflat = user_table.T.reshape(_FLAT)
    iflat = item_table.T.reshape(_FLAT)
    stash, pu = _sc_user(uid, uflat)
    return _sc_item(iid, iflat, stash, pu)


# speedup vs baseline: 1.2502x; 1.0256x over previous
import functools

import jax
import jax.numpy as jnp
from jax import lax
from jax.experimental import pallas as pl
from jax.experimental.pallas import tpu as pltpu
from jax.experimental.pallas import tpu_sc as plsc

_NUM_EMB = 90000
_EMB_DIM = 32
_BATCH = 16384
_FLAT = _NUM_EMB * _EMB_DIM

_info = plsc.get_sparse_core_info()
_NC = _info.num_cores
_NS = _info.num_subcores
_L = _info.num_lanes
_NW = _NC * _NS
_BPW = _BATCH // _NW
_ICHUNK = 128
_NICHUNK = _BPW // _ICHUNK
_WBUF = _EMB_DIM * _BPW

_MESH = plsc.VectorSubcoreMesh(core_axis_name="c", subcore_axis_name="s")
_PARAMS = pltpu.CompilerParams(needs_layout_passes=False)


def _stage_idx(idx_hbm, idx_v, base, sem):
    cp = pltpu.async_copy(idx_hbm.at[pl.ds(base, _BPW)], idx_v, sem)
    cp.wait()


def _gather_all(t_hbm, idx_v, buf_v, sem):
    def fire_dim(d, carry):
        src = t_hbm.at[pl.ds(d * _NUM_EMB, _NUM_EMB)]
        pltpu.async_copy(src.at[idx_v],
                         buf_v.at[pl.ds(d * _BPW, _BPW)], sem)
        return carry

    lax.fori_loop(0, _EMB_DIM, fire_dim, 0)
    pltpu.make_async_copy(t_hbm.at[pl.ds(0, _WBUF)], buf_v, sem).wait()


@functools.partial(
    pl.kernel, mesh=_MESH,
    out_type=jax.ShapeDtypeStruct((_NW * _WBUF,), jnp.float32),
    scratch_types=[
        pltpu.VMEM((_BPW,), jnp.int32),
        pltpu.VMEM((_WBUF,), jnp.float32),
        pltpu.SemaphoreType.DMA,
        pltpu.SemaphoreType.DMA,
    ],
    compiler_params=_PARAMS,
)
def _sc_user(uid_hbm, ut_hbm, stash_hbm, uidx_v, ubuf_v, sem_idx, sem_g):
    wid = lax.axis_index("s") * _NC + lax.axis_index("c")
    base = wid * _BPW
    _stage_idx(uid_hbm, uidx_v, base, sem_idx)
    _gather_all(ut_hbm, uidx_v, ubuf_v, sem_g)
    pltpu.sync_copy(ubuf_v, stash_hbm.at[pl.ds(wid * _WBUF, _WBUF)])


@functools.partial(
    pl.kernel, mesh=_MESH,
    out_type=jax.ShapeDtypeStruct((_BATCH,), jnp.float32),
    scratch_types=[
        pltpu.VMEM((_BPW,), jnp.int32),
        pltpu.VMEM((_WBUF,), jnp.float32),
        pltpu.VMEM((_WBUF,), jnp.float32),
        pltpu.VMEM((_BPW,), jnp.float32),
        pltpu.SemaphoreType.DMA,
        pltpu.SemaphoreType.DMA,
    ],
    compiler_params=_PARAMS,
)
def _sc_item(iid_hbm, it_hbm, stash_hbm, out_hbm,
             iidx_v, ibuf_v, ubuf_v, out_v, sem_idx, sem_g):
    wid = lax.axis_index("s") * _NC + lax.axis_index("c")
    base = wid * _BPW
    _stage_idx(iid_hbm, iidx_v, base, sem_idx)
    stash_cp = pltpu.async_copy(stash_hbm.at[pl.ds(wid * _WBUF, _WBUF)],
                                ubuf_v, sem_idx)
    _gather_all(it_hbm, iidx_v, ibuf_v, sem_g)
    stash_cp.wait()

    def group_body(g, carry):
        acc_d = jnp.zeros((_L,), jnp.float32)
        acc_u = jnp.zeros((_L,), jnp.float32)
        acc_i = jnp.zeros((_L,), jnp.float32)
        for d in range(_EMB_DIM):
            s = pl.ds(d * _BPW + g * _L, _L)
            ic = ibuf_v[s]
            uc = ubuf_v[s]
            acc_d = acc_d + uc * ic
            acc_u = acc_u + uc * uc
            acc_i = acc_i + ic * ic
        p = jnp.maximum(acc_u, 1e-16) * jnp.maximum(acc_i, 1e-16)
        seed = jnp.full((_L,), 0x5F3759DF, jnp.int32) - \
            lax.shift_right_logical(plsc.bitcast(p, jnp.int32), 1)
        y = plsc.bitcast(seed, jnp.float32)
        for _ in range(3):
            y = y * (1.5 - 0.5 * p * y * y)
        out_v[pl.ds(g * _L, _L)] = (6.0 * acc_d) * y
        return carry

    lax.fori_loop(0, _BPW // _L, group_body, 0)

    pltpu.sync_copy(out_v, out_hbm.at[pl.ds(base, _BPW)])


def kernel(user_id, item_id, user_table, item_table):
    uid = user_id.astype(jnp.int32)
    iid = item_id.astype(jnp.int32)
    uflat = user_table.T.reshape(_FLAT)
    iflat = item_table.T.reshape(_FLAT)
    stash = _sc_user(uid, uflat)
    return _sc_item(iid, iflat, stash)
